# baseline (speedup 1.0000x reference)
import jax
import jax.numpy as jnp
import numpy as np
from jax import lax
from jax.experimental import pallas as pl
from jax.experimental.pallas import tpu as pltpu
from jax.experimental.pallas import tpu_sc as plsc

_TOPK = 64
_N = 32768
_ROWS = 16 * 16 * 4
_L = 16
_NVEC = _N // _L
_NW = 32
_ROWS_PER_W = _ROWS // _NW
_NBKT = 256
_INT_MIN = np.int32(-(2 ** 31))


def _srl(x, k):
    return lax.shift_right_logical(x, jnp.full_like(x, k))


def _ukey(v):
    b = plsc.bitcast(v, jnp.int32)
    return b ^ (lax.shift_right_arithmetic(b, jnp.full_like(b, 31))
                | _INT_MIN)


def _skey(v):
    b = plsc.bitcast(v, jnp.int32)
    return b ^ (lax.shift_right_arithmetic(b, jnp.full_like(b, 31))
                & np.int32(0x7FFFFFFF))


def _body(x_hbm, out_hbm, rowa_v, rowb_v, ca_v, hist_v, sfx_v,
          sema, semb, semo):
    nc = 2
    wid = lax.axis_index("s") * nc + lax.axis_index("c")
    lanes = lax.iota(jnp.int32, _L)
    ones = jnp.ones((_L,), jnp.int32)
    zeros_i = jnp.zeros((_L,), jnp.int32)

    def io_slice(ref, row):
        return ref.at[row >> 6, (row >> 2) & 15, row & 3]

    def scan_bucket(r, beta):
        @plsc.parallel_loop(0, _NBKT, unroll=4, carry=zeros_i)
        def _cum(j, acc):
            b = beta(j)
            acc = acc + hist_v[pl.ds(b * _L, _L)]
            hist_v[pl.ds(b * _L, _L)] = zeros_i
            sfx_v[pl.ds(j * _L, _L)] = acc
            return acc

        def cum_at(o):
            return jnp.sum(sfx_v[pl.ds(o * _L, _L)])

        idx = jnp.int32(0)
        for step in (128, 64, 32, 16, 8, 4, 2, 1):
            t = idx + jnp.int32(step)
            idx = jnp.where(cum_at(t - 1) < r, t, idx)
        above = jnp.where(idx > 0, cum_at(jnp.maximum(idx - 1, 0)), 0)
        return idx, r - above

    def beta_raw(j):
        return jnp.where(j < 128, 127 - j, j)

    def beta_key(j):
        return _NBKT - 1 - j

    def process_row(li, row, buf, nbuf, sem_cur, sem_nxt):
        @pl.when(li + 1 < _ROWS_PER_W)
        def _():
            pltpu.async_copy(io_slice(x_hbm, row + 1), nbuf, sem_nxt)

        o1, r = scan_bucket(jnp.int32(_TOPK), beta_raw)
        b1_raw = beta_raw(o1)
        b1u = jnp.where(b1_raw < 128, b1_raw + 128, 255 - b1_raw)

        @pl.when(li > 0)
        def _():
            pltpu.make_async_copy(ca_v.at[pl.ds(0, _N)],
                                  io_slice(out_hbm, row - 1), semo).wait()

        @plsc.parallel_loop(0, _NVEC, unroll=16, carry=jnp.int32(0))
        def n1(i, w):
            v = buf[pl.ds(i * _L, _L)]
            u = _ukey(v)
            msk_ge = _srl(u, 24) >= b1u
            plsc.store_compressed(ca_v.at[pl.ds(w, _L)],
                                  plsc.bitcast(u, jnp.float32), mask=msk_ge)
            return w + plsc.all_reduce_population_count(msk_ge)[0]

        nv1 = jnp.right_shift(n1 + _L - 1, 4)

        @plsc.parallel_loop(0, nv1)
        def _(i):
            u = plsc.bitcast(ca_v[pl.ds(i * _L, _L)], jnp.int32)
            valid = (i * _L + lanes) < n1
            msk = valid & (_srl(u, 24) == b1u)
            bk16 = _srl(u, 12) & 0xFF0
            plsc.addupdate_scatter(hist_v, [bk16 + lanes], ones, mask=msk)

        o2, r = scan_bucket(r, beta_key)
        b2 = beta_key(o2)
        hi16 = (b1u << 8) | b2

        @plsc.parallel_loop(0, nv1)
        def _(i):
            u = plsc.bitcast(ca_v[pl.ds(i * _L, _L)], jnp.int32)
            valid = (i * _L + lanes) < n1
            msk = valid & (_srl(u, 16) == hi16)
            bk16 = _srl(u, 4) & 0xFF0
            plsc.addupdate_scatter(hist_v, [bk16 + lanes], ones, mask=msk)

        o3, r = scan_bucket(r, beta_key)
        b3 = beta_key(o3)
        hi24 = (hi16 << 8) | b3

        @plsc.parallel_loop(0, nv1)
        def _(i):
            u = plsc.bitcast(ca_v[pl.ds(i * _L, _L)], jnp.int32)
            valid = (i * _L + lanes) < n1
            msk = valid & (_srl(u, 8) == hi24)
            plsc.addupdate_scatter(hist_v, [((u & 0xFF) << 4) + lanes], ones,
                                   mask=msk)

        o4, _unused = scan_bucket(r, beta_key)
        b4 = beta_key(o4)

        t_s = ((b1u << 24) | (b2 << 16) | (b3 << 8) | b4) ^ _INT_MIN

        def inv_key(u):
            ks = u ^ _INT_MIN
            bits = jnp.where(ks >= 0, ks, ks ^ 0x7FFFFFFF)
            return plsc.bitcast(bits, jnp.float32), ks

        @plsc.parallel_loop(0, nv1,
                            carry=jnp.full((_L,), -jnp.inf, jnp.float32))
        def mx(i, acc):
            u = plsc.bitcast(ca_v[pl.ds(i * _L, _L)], jnp.int32)
            valid = (i * _L + lanes) < n1
            v, _ = inv_key(u)
            return jnp.maximum(acc, jnp.where(valid, v, -jnp.inf))
        m = jnp.max(mx)

        @plsc.parallel_loop(0, nv1, carry=jnp.zeros((_L,), jnp.float32))
        def sv(i, acc):
            u = plsc.bitcast(ca_v[pl.ds(i * _L, _L)], jnp.int32)
            valid = (i * _L + lanes) < n1
            v, ks = inv_key(u)
            return acc + jnp.where(valid & (ks >= t_s), jnp.exp(v - m), 0.0)
        inv = jnp.ones((_L,), jnp.float32) / jnp.broadcast_to(jnp.sum(sv),
                                                              (_L,))

        @pl.when(li + 1 < _ROWS_PER_W)
        def _():
            pltpu.make_async_copy(io_slice(x_hbm, row + 1), nbuf,
                                  sem_nxt).wait()

        @plsc.parallel_loop(0, _NVEC, unroll=8)
        def _(i):
            v = buf[pl.ds(i * _L, _L)]
            ks = _skey(v)
            ca_v[pl.ds(i * _L, _L)] = jnp.where(ks >= t_s,
                                                jnp.exp(v - m) * inv, 0.0)
            nv = nbuf[pl.ds(i * _L, _L)]
            bk16 = _srl(plsc.bitcast(nv, jnp.int32), 20) & 0xFF0
            plsc.addupdate_scatter(hist_v, [bk16 + lanes], ones)

        pltpu.async_copy(ca_v.at[pl.ds(0, _N)], io_slice(out_hbm, row), semo)

    row0 = wid * _ROWS_PER_W
    pltpu.async_copy(io_slice(x_hbm, row0), rowa_v, sema)

    @plsc.parallel_loop(0, _NBKT, unroll=8)
    def _(i):
        hist_v[pl.ds(i * _L, _L)] = zeros_i

    pltpu.make_async_copy(io_slice(x_hbm, row0), rowa_v, sema).wait()

    @plsc.parallel_loop(0, _NVEC, unroll=8)
    def _(i):
        v = rowa_v[pl.ds(i * _L, _L)]
        bk16 = _srl(plsc.bitcast(v, jnp.int32), 20) & 0xFF0
        plsc.addupdate_scatter(hist_v, [bk16 + lanes], ones)

    def pair_loop(p, c):
        process_row(2 * p, row0 + 2 * p, rowa_v, rowb_v, sema, semb)
        process_row(2 * p + 1, row0 + 2 * p + 1, rowb_v, rowa_v, semb, sema)
        return c
    lax.fori_loop(0, _ROWS_PER_W // 2, pair_loop, 0)

    last = row0 + _ROWS_PER_W - 1
    pltpu.make_async_copy(ca_v.at[pl.ds(0, _N)],
                          io_slice(out_hbm, last), semo).wait()


def kernel(qk_dots):
    call = pl.kernel(
        _body,
        out_type=jax.ShapeDtypeStruct(qk_dots.shape, jnp.float32),
        mesh=plsc.VectorSubcoreMesh(core_axis_name="c", subcore_axis_name="s"),
        compiler_params=pltpu.CompilerParams(needs_layout_passes=False),
        scratch_types=[
            pltpu.VMEM((_N,), jnp.float32),
            pltpu.VMEM((_N,), jnp.float32),
            pltpu.VMEM((_N + _L,), jnp.float32),
            pltpu.VMEM((_NBKT * _L,), jnp.int32),
            pltpu.VMEM((_NBKT * _L,), jnp.int32),
            pltpu.SemaphoreType.DMA,
            pltpu.SemaphoreType.DMA,
            pltpu.SemaphoreType.DMA,
        ],
    )
    return call(qk_dots)


# speedup vs baseline: 1.0830x; 1.0830x over previous
import jax
import jax.numpy as jnp
import numpy as np
from jax import lax
from jax.experimental import pallas as pl
from jax.experimental.pallas import tpu as pltpu
from jax.experimental.pallas import tpu_sc as plsc

_TOPK = 64
_N = 32768
_ROWS = 16 * 16 * 4
_L = 16
_NVEC = _N // _L
_NW = 32
_ROWS_PER_W = _ROWS // _NW
_NBKT = 256
_INT_MIN = np.int32(-(2 ** 31))


def _srl(x, k):
    return lax.shift_right_logical(x, jnp.full_like(x, k))


def _ukey(v):
    b = plsc.bitcast(v, jnp.int32)
    return b ^ (lax.shift_right_arithmetic(b, jnp.full_like(b, 31))
                | _INT_MIN)


def _skey(v):
    b = plsc.bitcast(v, jnp.int32)
    return b ^ (lax.shift_right_arithmetic(b, jnp.full_like(b, 31))
                & np.int32(0x7FFFFFFF))


def _body(x_hbm, out_hbm, rowa_v, rowb_v, ca_v, hist_v, sfx_v,
          sema, semb, semo):
    nc = 2
    wid = lax.axis_index("s") * nc + lax.axis_index("c")
    lanes = lax.iota(jnp.int32, _L)
    ones = jnp.ones((_L,), jnp.int32)
    zeros_i = jnp.zeros((_L,), jnp.int32)

    def io_slice(ref, row):
        return ref.at[row >> 6, (row >> 2) & 15, row & 3]

    def scan_bucket(r, beta):
        @plsc.parallel_loop(0, _NBKT, unroll=4, carry=zeros_i)
        def _cum(j, acc):
            b = beta(j)
            acc = acc + hist_v[pl.ds(b * _L, _L)]
            hist_v[pl.ds(b * _L, _L)] = zeros_i
            sfx_v[pl.ds(j * _L, _L)] = acc
            return acc

        def cum_at(o):
            return jnp.sum(sfx_v[pl.ds(o * _L, _L)])

        idx = jnp.int32(0)
        for step in (128, 64, 32, 16, 8, 4, 2, 1):
            t = idx + jnp.int32(step)
            idx = jnp.where(cum_at(t - 1) < r, t, idx)
        above = jnp.where(idx > 0, cum_at(jnp.maximum(idx - 1, 0)), 0)
        return idx, r - above

    def beta_raw(j):
        return jnp.where(j < 128, 127 - j, j)

    def beta_key(j):
        return _NBKT - 1 - j

    def process_row(li, row, buf, nbuf, sem_cur, sem_nxt):
        @pl.when(li + 1 < _ROWS_PER_W)
        def _():
            pltpu.async_copy(io_slice(x_hbm, row + 1), nbuf, sem_nxt)

        o1, r = scan_bucket(jnp.int32(_TOPK), beta_raw)
        b1_raw = beta_raw(o1)
        b1u = jnp.where(b1_raw < 128, b1_raw + 128, 255 - b1_raw)

        @pl.when(li > 0)
        def _():
            pltpu.make_async_copy(ca_v.at[pl.ds(0, _N)],
                                  io_slice(out_hbm, row - 1), semo).wait()

        @plsc.parallel_loop(0, _NVEC, unroll=8, carry=jnp.int32(0))
        def n1(i, w):
            v = buf[pl.ds(i * _L, _L)]
            u = _ukey(v)
            msk_ge = _srl(u, 24) >= b1u
            plsc.store_compressed(ca_v.at[pl.ds(w, _L)],
                                  plsc.bitcast(u, jnp.float32), mask=msk_ge)
            return w + plsc.all_reduce_population_count(msk_ge)[0]

        nv1 = jnp.right_shift(n1 + _L - 1, 4)

        @plsc.parallel_loop(0, nv1)
        def _(i):
            u = plsc.bitcast(ca_v[pl.ds(i * _L, _L)], jnp.int32)
            valid = (i * _L + lanes) < n1
            msk = valid & (_srl(u, 24) == b1u)
            bk16 = _srl(u, 12) & 0xFF0
            plsc.addupdate_scatter(hist_v, [bk16 + lanes], ones, mask=msk)

        o2, r = scan_bucket(r, beta_key)
        b2 = beta_key(o2)
        hi16 = (b1u << 8) | b2

        @plsc.parallel_loop(0, nv1)
        def _(i):
            u = plsc.bitcast(ca_v[pl.ds(i * _L, _L)], jnp.int32)
            valid = (i * _L + lanes) < n1
            msk = valid & (_srl(u, 16) == hi16)
            bk16 = _srl(u, 4) & 0xFF0
            plsc.addupdate_scatter(hist_v, [bk16 + lanes], ones, mask=msk)

        o3, r = scan_bucket(r, beta_key)
        b3 = beta_key(o3)
        hi24 = (hi16 << 8) | b3

        @plsc.parallel_loop(0, nv1)
        def _(i):
            u = plsc.bitcast(ca_v[pl.ds(i * _L, _L)], jnp.int32)
            valid = (i * _L + lanes) < n1
            msk = valid & (_srl(u, 8) == hi24)
            plsc.addupdate_scatter(hist_v, [((u & 0xFF) << 4) + lanes], ones,
                                   mask=msk)

        o4, _unused = scan_bucket(r, beta_key)
        b4 = beta_key(o4)

        t_s = ((b1u << 24) | (b2 << 16) | (b3 << 8) | b4) ^ _INT_MIN

        def inv_key(u):
            ks = u ^ _INT_MIN
            bits = jnp.where(ks >= 0, ks, ks ^ 0x7FFFFFFF)
            return plsc.bitcast(bits, jnp.float32), ks

        @plsc.parallel_loop(0, nv1,
                            carry=jnp.full((_L,), -jnp.inf, jnp.float32))
        def mx(i, acc):
            u = plsc.bitcast(ca_v[pl.ds(i * _L, _L)], jnp.int32)
            valid = (i * _L + lanes) < n1
            v, _ = inv_key(u)
            return jnp.maximum(acc, jnp.where(valid, v, -jnp.inf))
        m = jnp.max(mx)

        @plsc.parallel_loop(0, nv1, carry=jnp.zeros((_L,), jnp.float32))
        def sv(i, acc):
            u = plsc.bitcast(ca_v[pl.ds(i * _L, _L)], jnp.int32)
            valid = (i * _L + lanes) < n1
            v, ks = inv_key(u)
            return acc + jnp.where(valid & (ks >= t_s), jnp.exp(v - m), 0.0)
        inv = jnp.ones((_L,), jnp.float32) / jnp.broadcast_to(jnp.sum(sv),
                                                              (_L,))

        @pl.when(li + 1 < _ROWS_PER_W)
        def _():
            pltpu.make_async_copy(io_slice(x_hbm, row + 1), nbuf,
                                  sem_nxt).wait()

        for c in range(4):
            @plsc.parallel_loop(c * (_NVEC // 4), (c + 1) * (_NVEC // 4),
                                unroll=8)
            def _(i):
                v = buf[pl.ds(i * _L, _L)]
                ks = _skey(v)
                ca_v[pl.ds(i * _L, _L)] = jnp.where(ks >= t_s,
                                                    jnp.exp(v - m) * inv, 0.0)
                nv = nbuf[pl.ds(i * _L, _L)]
                bk16 = _srl(plsc.bitcast(nv, jnp.int32), 20) & 0xFF0
                plsc.addupdate_scatter(hist_v, [bk16 + lanes], ones)

            pltpu.async_copy(
                ca_v.at[pl.ds(c * (_N // 4), _N // 4)],
                out_hbm.at[row >> 6, (row >> 2) & 15, row & 3,
                           pl.ds(c * (_N // 4), _N // 4)], semo)

    row0 = wid * _ROWS_PER_W
    pltpu.async_copy(io_slice(x_hbm, row0), rowa_v, sema)

    @plsc.parallel_loop(0, _NBKT, unroll=8)
    def _(i):
        hist_v[pl.ds(i * _L, _L)] = zeros_i

    pltpu.make_async_copy(io_slice(x_hbm, row0), rowa_v, sema).wait()

    @plsc.parallel_loop(0, _NVEC, unroll=8)
    def _(i):
        v = rowa_v[pl.ds(i * _L, _L)]
        bk16 = _srl(plsc.bitcast(v, jnp.int32), 20) & 0xFF0
        plsc.addupdate_scatter(hist_v, [bk16 + lanes], ones)

    def pair_loop(p, c):
        process_row(2 * p, row0 + 2 * p, rowa_v, rowb_v, sema, semb)
        process_row(2 * p + 1, row0 + 2 * p + 1, rowb_v, rowa_v, semb, sema)
        return c
    lax.fori_loop(0, _ROWS_PER_W // 2, pair_loop, 0)

    last = row0 + _ROWS_PER_W - 1
    pltpu.make_async_copy(ca_v.at[pl.ds(0, _N)],
                          io_slice(out_hbm, last), semo).wait()


def kernel(qk_dots):
    call = pl.kernel(
        _body,
        out_type=jax.ShapeDtypeStruct(qk_dots.shape, jnp.float32),
        mesh=plsc.VectorSubcoreMesh(core_axis_name="c", subcore_axis_name="s"),
        compiler_params=pltpu.CompilerParams(needs_layout_passes=False),
        scratch_types=[
            pltpu.VMEM((_N,), jnp.float32),
            pltpu.VMEM((_N,), jnp.float32),
            pltpu.VMEM((_N + _L,), jnp.float32),
            pltpu.VMEM((_NBKT * _L,), jnp.int32),
            pltpu.VMEM((_NBKT * _L,), jnp.int32),
            pltpu.SemaphoreType.DMA,
            pltpu.SemaphoreType.DMA,
            pltpu.SemaphoreType.DMA,
        ],
    )
    return call(qk_dots)
